# Initial kernel scaffold; baseline (speedup 1.0000x reference)
import jax
import jax.numpy as jnp
from jax.experimental import pallas as pl


def kernel(token_ids, Embedding_Matrix):
    raise NotImplementedError("write your pallas kernel here")


# speedup vs baseline: 1.6847x; 1.6847x over previous
import functools

import jax
import jax.numpy as jnp
from jax import lax
from jax.experimental import pallas as pl
from jax.experimental.pallas import tpu as pltpu
from jax.experimental.pallas import tpu_sc as plsc

NUM_CORES = 2
NUM_SUBCORES = 16
NUM_WORKERS = NUM_CORES * NUM_SUBCORES

CHUNK = 128


@functools.partial(jax.jit, static_argnums=(2, 3))
def _gather_sc(table, idx2d, n_rows, d):
    n_chunks_total = idx2d.shape[0]
    chunks_per_w = n_chunks_total // NUM_WORKERS
    rows_per_w = chunks_per_w * CHUNK

    mesh = plsc.VectorSubcoreMesh(
        core_axis_name="c", subcore_axis_name="s",
        num_cores=NUM_CORES, num_subcores=NUM_SUBCORES)

    @functools.partial(
        pl.kernel,
        mesh=mesh,
        out_type=jax.ShapeDtypeStruct((n_rows, d), jnp.float32),
        compiler_params=pltpu.CompilerParams(use_tc_tiling_on_sc=False),
        scratch_types=[
            pltpu.VMEM((chunks_per_w, CHUNK), jnp.int32),
            pltpu.VMEM((CHUNK, d), jnp.float32),
            pltpu.SemaphoreType.DMA,
        ],
    )
    def body(table_hbm, idx_hbm, out_hbm, idx_v, rows_v, sem):
        wid = lax.axis_index("s") * NUM_CORES + lax.axis_index("c")
        chunk_base = wid * chunks_per_w
        row_base = wid * rows_per_w
        pltpu.sync_copy(idx_hbm.at[pl.ds(chunk_base, chunks_per_w)], idx_v)

        def step(i, carry):
            pltpu.async_copy(table_hbm.at[idx_v.at[i]], rows_v, sem).wait()
            pltpu.sync_copy(rows_v, out_hbm.at[pl.ds(row_base + i * CHUNK, CHUNK)])
            return carry

        lax.fori_loop(0, chunks_per_w, step, 0)

    return body(table, idx2d)


def kernel(token_ids, Embedding_Matrix):
    n_tok, n_per = token_ids.shape
    d = Embedding_Matrix.shape[1]
    n_rows = n_tok * n_per
    idx2d = token_ids.reshape(n_rows // CHUNK, CHUNK).astype(jnp.int32)
    out = _gather_sc(Embedding_Matrix, idx2d, n_rows, d)
    return out.reshape(n_tok, n_per, d)


# speedup vs baseline: 1.8703x; 1.1101x over previous
import functools

import jax
import jax.numpy as jnp
from jax import lax
from jax.experimental import pallas as pl
from jax.experimental.pallas import tpu as pltpu
from jax.experimental.pallas import tpu_sc as plsc

NUM_CORES = 2
NUM_SUBCORES = 16
NUM_WORKERS = NUM_CORES * NUM_SUBCORES

CHUNK = 128
K = 4
ROWS = K * CHUNK


@functools.partial(jax.jit, static_argnums=(2, 3))
def _gather_sc(table, idx2d, n_rows, d):
    n_chunks_total = idx2d.shape[0]
    chunks_per_w = n_chunks_total // NUM_WORKERS
    rows_per_w = chunks_per_w * CHUNK
    n_steps = chunks_per_w // K
    assert n_steps % 2 == 0

    mesh = plsc.VectorSubcoreMesh(
        core_axis_name="c", subcore_axis_name="s",
        num_cores=NUM_CORES, num_subcores=NUM_SUBCORES)

    @functools.partial(
        pl.kernel,
        mesh=mesh,
        out_type=jax.ShapeDtypeStruct((n_rows, d), jnp.float32),
        compiler_params=pltpu.CompilerParams(use_tc_tiling_on_sc=False),
        scratch_types=[
            pltpu.VMEM((chunks_per_w, CHUNK), jnp.int32),
            pltpu.VMEM((2, ROWS, d), jnp.float32),
            pltpu.SemaphoreType.DMA,
            pltpu.SemaphoreType.DMA,
            pltpu.SemaphoreType.DMA,
            pltpu.SemaphoreType.DMA,
        ],
    )
    def body(table_hbm, idx_hbm, out_hbm, idx_v, rows_v, g0, g1, w0, w1):
        wid = lax.axis_index("s") * NUM_CORES + lax.axis_index("c")
        chunk_base = wid * chunks_per_w
        row_base = wid * rows_per_w
        gsem = (g0, g1)
        wsem = (w0, w1)
        pltpu.sync_copy(idx_hbm.at[pl.ds(chunk_base, chunks_per_w)], idx_v)

        def fire_gathers(s, b):
            for j in range(K):
                pltpu.async_copy(
                    table_hbm.at[idx_v.at[s * K + j]],
                    rows_v.at[b].at[pl.ds(j * CHUNK, CHUNK)],
                    gsem[b])

        def wait_gathers(b):
            pltpu.make_async_copy(
                out_hbm.at[pl.ds(0, ROWS)], rows_v.at[b], gsem[b]).wait()

        def fire_write(s, b):
            pltpu.async_copy(
                rows_v.at[b],
                out_hbm.at[pl.ds(row_base + s * ROWS, ROWS)],
                wsem[b])

        def wait_write(b):
            pltpu.make_async_copy(
                rows_v.at[b], out_hbm.at[pl.ds(row_base, ROWS)], wsem[b]).wait()

        fire_gathers(0, 0)

        @pl.loop(0, n_steps, step=2)
        def _steps(t):
            for b in range(2):
                s = t + b
                b2 = 1 - b
                wait_gathers(b)
                fire_write(s, b)

                @pl.when(s + 1 < n_steps)
                def _prefetch():
                    @pl.when(s >= 1)
                    def _drain():
                        wait_write(b2)
                    fire_gathers(s + 1, b2)

        wait_write(0)
        wait_write(1)

    return body(table, idx2d)


def kernel(token_ids, Embedding_Matrix):
    n_tok, n_per = token_ids.shape
    d = Embedding_Matrix.shape[1]
    n_rows = n_tok * n_per
    idx2d = token_ids.reshape(n_rows // CHUNK, CHUNK).astype(jnp.int32)
    out = _gather_sc(Embedding_Matrix, idx2d, n_rows, d)
    return out.reshape(n_tok, n_per, d)


# speedup vs baseline: 1.8765x; 1.0033x over previous
import functools

import jax
import jax.numpy as jnp
from jax import lax
from jax.experimental import pallas as pl
from jax.experimental.pallas import tpu as pltpu
from jax.experimental.pallas import tpu_sc as plsc

NUM_CORES = 2
NUM_SUBCORES = 16
NUM_WORKERS = NUM_CORES * NUM_SUBCORES

CHUNK = 128
K = 5
ROWS = K * CHUNK


@functools.partial(jax.jit, static_argnums=(2, 3))
def _gather_sc(table, idx2d, n_rows, d):
    n_chunks_total = idx2d.shape[0]
    chunks_per_w = n_chunks_total // NUM_WORKERS
    rows_per_w = chunks_per_w * CHUNK
    n_steps = chunks_per_w // K
    assert n_steps % 2 == 0

    mesh = plsc.VectorSubcoreMesh(
        core_axis_name="c", subcore_axis_name="s",
        num_cores=NUM_CORES, num_subcores=NUM_SUBCORES)

    @functools.partial(
        pl.kernel,
        mesh=mesh,
        out_type=jax.ShapeDtypeStruct((n_rows, d), jnp.float32),
        compiler_params=pltpu.CompilerParams(use_tc_tiling_on_sc=False),
        scratch_types=[
            pltpu.VMEM((chunks_per_w, CHUNK), jnp.int32),
            pltpu.VMEM((2, ROWS, d), jnp.float32),
            pltpu.SemaphoreType.DMA,
            pltpu.SemaphoreType.DMA,
            pltpu.SemaphoreType.DMA,
            pltpu.SemaphoreType.DMA,
        ],
    )
    def body(table_hbm, idx_hbm, out_hbm, idx_v, rows_v, g0, g1, w0, w1):
        wid = lax.axis_index("s") * NUM_CORES + lax.axis_index("c")
        chunk_base = wid * chunks_per_w
        row_base = wid * rows_per_w
        gsem = (g0, g1)
        wsem = (w0, w1)
        pltpu.sync_copy(idx_hbm.at[pl.ds(chunk_base, chunks_per_w)], idx_v)

        def fire_gathers(s, b):
            for j in range(K):
                pltpu.async_copy(
                    table_hbm.at[idx_v.at[s * K + j]],
                    rows_v.at[b].at[pl.ds(j * CHUNK, CHUNK)],
                    gsem[b])

        def wait_gathers(b):
            pltpu.make_async_copy(
                out_hbm.at[pl.ds(0, ROWS)], rows_v.at[b], gsem[b]).wait()

        def fire_write(s, b):
            pltpu.async_copy(
                rows_v.at[b],
                out_hbm.at[pl.ds(row_base + s * ROWS, ROWS)],
                wsem[b])

        def wait_write(b):
            pltpu.make_async_copy(
                rows_v.at[b], out_hbm.at[pl.ds(row_base, ROWS)], wsem[b]).wait()

        fire_gathers(0, 0)

        @pl.loop(0, n_steps, step=2)
        def _steps(t):
            for b in range(2):
                s = t + b
                b2 = 1 - b

                @pl.when(s + 1 < n_steps)
                def _prefetch():
                    @pl.when(s >= 1)
                    def _drain():
                        wait_write(b2)
                    fire_gathers(s + 1, b2)

                wait_gathers(b)
                fire_write(s, b)

        wait_write(0)
        wait_write(1)

    return body(table, idx2d)


def kernel(token_ids, Embedding_Matrix):
    n_tok, n_per = token_ids.shape
    d = Embedding_Matrix.shape[1]
    n_rows = n_tok * n_per
    idx2d = token_ids.reshape(n_rows // CHUNK, CHUNK).astype(jnp.int32)
    out = _gather_sc(Embedding_Matrix, idx2d, n_rows, d)
    return out.reshape(n_tok, n_per, d)
